# baseline (speedup 1.0000x reference)
import functools

import jax
import jax.numpy as jnp
from jax.experimental import pallas as pl
from jax.experimental.pallas import tpu as pltpu

_SEL_RANKS = frozenset((7, 4, 16, 19, 2, 5, 3, 6, 18, 10))

_RB = 8
_K = 20
_NEG = 10
_MASKV = -1000000000.0


def _body(rowid_ref, *refs, S, P):
    sims = refs[:_RB]
    j_ref, e_ref, m_ref = refs[_RB:_RB + 3]
    o_tot, o_c, o_f = refs[_RB + 3:_RB + 6]
    acc = refs[_RB + 6]

    p = pl.program_id(0)

    rows = jnp.concatenate(
        [jnp.reshape(s[...], (1, S)) for s in sims], axis=0)
    jv = j_ref[...]
    iota = jax.lax.broadcasted_iota(jnp.int32, (_RB, S), 1)
    isj = iota == jv
    pos = jnp.sum(jnp.where(isj, rows, 0.0), axis=1, keepdims=True)
    x = jnp.where(isj, _MASKV, rows)

    hinge = jnp.zeros((_RB, 1), jnp.float32)
    for r in range(_K):
        m = jnp.max(x, axis=1, keepdims=True)
        if r in _SEL_RANKS:
            v = jnp.where(m == _MASKV, pos, m)
            hinge += jnp.maximum(1.0 - pos + v, 0.0)
        if r < _K - 1:
            idx = jnp.min(jnp.where(x == m, iota, S), axis=1, keepdims=True)
            x = jnp.where(iota == idx, -jnp.inf, x)

    part = jnp.sum(hinge)

    @pl.when(p == 0)
    def _init():
        acc[0] = 0.0

    acc[0] += part

    @pl.when(p == pl.num_programs(0) - 1)
    def _fin():
        loss_c = acc[0] / (P * float(_NEG))
        e = e_ref[...]
        w = 1.0 / jnp.clip(e[2:3, :], 0.0001, None)
        per = w * (e[0:1, :] * e[0:1, :] + e[1:2, :] * e[1:2, :])
        mk = m_ref[...]
        loss_f = jnp.sum(per * mk) / jnp.maximum(jnp.sum(mk), 1.0)
        o_tot[...] = jnp.reshape(1.0 * loss_c + 0.5 * loss_f, (1, 1))
        o_c[...] = jnp.reshape(loss_c, (1, 1))
        o_f[...] = jnp.reshape(loss_f, (1, 1))


def kernel(sim_matrix, spv_b_ids, spv_i_ids, spv_j_ids, expec_f, gt_mask):
    B, L, S = sim_matrix.shape
    P = spv_b_ids.shape[0]
    sim3d = sim_matrix.reshape(B * L, 1, S)
    rowid = (spv_b_ids.astype(jnp.int32) * L + spv_i_ids.astype(jnp.int32))
    jcol = spv_j_ids.astype(jnp.int32).reshape(P, 1)
    expec_t = expec_f.astype(jnp.float32).T
    maskf = gt_mask.astype(jnp.float32).reshape(1, P)

    grid = (P // _RB,)
    sim_spec = [
        pl.BlockSpec((1, 1, S), functools.partial(
            lambda gp, rid, r=0: (rid[_RB * gp + r], 0, 0), r=r))
        for r in range(_RB)
    ]
    in_specs = sim_spec + [
        pl.BlockSpec((_RB, 1), lambda gp, rid: (gp, 0)),
        pl.BlockSpec((3, P), lambda gp, rid: (0, 0)),
        pl.BlockSpec((1, P), lambda gp, rid: (0, 0)),
    ]
    out_specs = [pl.BlockSpec((1, 1), lambda gp, rid: (0, 0))] * 3

    grid_spec = pltpu.PrefetchScalarGridSpec(
        num_scalar_prefetch=1,
        grid=grid,
        in_specs=in_specs,
        out_specs=out_specs,
        scratch_shapes=[pltpu.SMEM((1,), jnp.float32)],
    )
    out_shape = [jax.ShapeDtypeStruct((1, 1), jnp.float32)] * 3

    tot, lc, lf = pl.pallas_call(
        functools.partial(_body, S=S, P=P),
        grid_spec=grid_spec,
        out_shape=out_shape,
        compiler_params=pltpu.CompilerParams(
            dimension_semantics=("arbitrary",)),
    )(rowid, *([sim3d] * _RB), jcol, expec_t, maskf)

    return (tot[0, 0],
            jax.lax.stop_gradient(lc[0, 0]),
            jax.lax.stop_gradient(lf[0, 0]))


# speedup vs baseline: 2.3816x; 2.3816x over previous
import functools

import jax
import jax.numpy as jnp
from jax.experimental import pallas as pl
from jax.experimental.pallas import tpu as pltpu

_SEL_RANKS = frozenset((7, 4, 16, 19, 2, 5, 3, 6, 18, 10))

_RB = 16
_K = 20
_NEG = 10
_MASKV = -1000000000.0
_NINF = float("-inf")
_D = 4
_G = 4


def _slow_hinge(x, pos, iota, S):
    hinge = jnp.zeros_like(pos)
    for r in range(_K):
        m = jnp.max(x, axis=1, keepdims=True)
        if r in _SEL_RANKS:
            v = jnp.where(m == _MASKV, pos, m)
            hinge += jnp.maximum(1.0 - pos + v, 0.0)
        if r < _K - 1:
            idx = jnp.min(jnp.where(x == m, iota, S), axis=1, keepdims=True)
            x = jnp.where(iota == idx, -jnp.inf, x)
    return hinge


def _body(rowid_ref, *refs, S, P):
    sims = refs[:_RB]
    j_ref, e_ref, m_ref = refs[_RB:_RB + 3]
    o_tot, o_c, o_f = refs[_RB + 3:_RB + 6]
    acc = refs[_RB + 6]

    p = pl.program_id(0)

    rows = jnp.concatenate(
        [jnp.reshape(s[...], (1, S)) for s in sims], axis=0)
    jv = j_ref[...]
    iota = jax.lax.broadcasted_iota(jnp.int32, (_RB, S), 1)
    isj = iota == jv
    pos = jnp.sum(jnp.where(isj, rows, 0.0), axis=1, keepdims=True)
    x = jnp.where(isj, _MASKV, rows)

    nchunks = (S + 127) // 128
    per_g = (nchunks + _G - 1) // _G
    stacks = [[jnp.full((_RB, 128), _NINF, jnp.float32)
               for _ in range(_D)] for _ in range(_G)]
    for g in range(_G):
        for q in range(per_g):
            c0 = (g * per_g + q) * 128
            if c0 >= S:
                break
            w = min(128, S - c0)
            c = x[:, c0:c0 + w]
            if w < 128:
                c = jnp.concatenate(
                    [c, jnp.full((_RB, 128 - w), _NINF, jnp.float32)], axis=1)
            st = stacks[g]
            for d in range(_D):
                hi = jnp.maximum(st[d], c)
                c = jnp.minimum(st[d], c)
                st[d] = hi

    ms = []
    for r in range(_K):
        top = stacks[0][0]
        for g in range(1, _G):
            top = jnp.maximum(top, stacks[g][0])
        m = jnp.max(top, axis=1, keepdims=True)
        ms.append(m)
        if r < _K - 1:
            for g in range(_G):
                st = stacks[g]
                hit = st[0] == m
                for d in range(_D - 1):
                    st[d] = jnp.where(hit, st[d + 1], st[d])
                st[_D - 1] = jnp.where(hit, _NINF, st[_D - 1])

    n = jnp.sum((x >= ms[_K - 1]).astype(jnp.float32), axis=1, keepdims=True)
    ok = jnp.all(n == float(_K))

    def _fast(_):
        h = jnp.zeros((_RB, 1), jnp.float32)
        for r in sorted(_SEL_RANKS):
            v = jnp.where(ms[r] == _MASKV, pos, ms[r])
            h += jnp.maximum(1.0 - pos + v, 0.0)
        return h

    hinge = jax.lax.cond(ok, _fast, lambda _: _slow_hinge(x, pos, iota, S), 0)
    part = jnp.sum(hinge)

    @pl.when(p == 0)
    def _init():
        acc[0] = 0.0

    acc[0] += part

    @pl.when(p == pl.num_programs(0) - 1)
    def _fin():
        loss_c = acc[0] / (P * float(_NEG))
        e = e_ref[...]
        w = 1.0 / jnp.clip(e[2:3, :], 0.0001, None)
        per = w * (e[0:1, :] * e[0:1, :] + e[1:2, :] * e[1:2, :])
        mk = m_ref[...]
        loss_f = jnp.sum(per * mk) / jnp.maximum(jnp.sum(mk), 1.0)
        o_tot[...] = jnp.reshape(1.0 * loss_c + 0.5 * loss_f, (1, 1))
        o_c[...] = jnp.reshape(loss_c, (1, 1))
        o_f[...] = jnp.reshape(loss_f, (1, 1))


def kernel(sim_matrix, spv_b_ids, spv_i_ids, spv_j_ids, expec_f, gt_mask):
    B, L, S = sim_matrix.shape
    P = spv_b_ids.shape[0]
    sim3d = sim_matrix.reshape(B * L, 1, S)
    rowid = (spv_b_ids.astype(jnp.int32) * L + spv_i_ids.astype(jnp.int32))
    jcol = spv_j_ids.astype(jnp.int32).reshape(P, 1)
    expec_t = expec_f.astype(jnp.float32).T
    maskf = gt_mask.astype(jnp.float32).reshape(1, P)

    grid = (P // _RB,)
    sim_spec = [
        pl.BlockSpec((1, 1, S), functools.partial(
            lambda gp, rid, r=0: (rid[_RB * gp + r], 0, 0), r=r))
        for r in range(_RB)
    ]
    in_specs = sim_spec + [
        pl.BlockSpec((_RB, 1), lambda gp, rid: (gp, 0)),
        pl.BlockSpec((3, P), lambda gp, rid: (0, 0)),
        pl.BlockSpec((1, P), lambda gp, rid: (0, 0)),
    ]
    out_specs = [pl.BlockSpec((1, 1), lambda gp, rid: (0, 0))] * 3

    grid_spec = pltpu.PrefetchScalarGridSpec(
        num_scalar_prefetch=1,
        grid=grid,
        in_specs=in_specs,
        out_specs=out_specs,
        scratch_shapes=[pltpu.SMEM((1,), jnp.float32)],
    )
    out_shape = [jax.ShapeDtypeStruct((1, 1), jnp.float32)] * 3

    tot, lc, lf = pl.pallas_call(
        functools.partial(_body, S=S, P=P),
        grid_spec=grid_spec,
        out_shape=out_shape,
        compiler_params=pltpu.CompilerParams(
            dimension_semantics=("arbitrary",)),
    )(rowid, *([sim3d] * _RB), jcol, expec_t, maskf)

    return (tot[0, 0],
            jax.lax.stop_gradient(lc[0, 0]),
            jax.lax.stop_gradient(lf[0, 0]))
